# baseline (speedup 1.0000x reference)
import functools

import jax
import jax.numpy as jnp
from jax import lax
from jax.experimental import pallas as pl
from jax.experimental.pallas import tpu as pltpu
from jax.experimental.pallas import tpu_sc as plsc

S = 2048
H = 12
D = 768
K = 1228
OUT_ROWS = K + 1
P_W = 1280
N_FULL_W = 30
ROWS_PER_W = 40
TAIL_BASE = N_FULL_W * ROWS_PER_W
TAIL_A = 24
TAIL_B = 8
TAIL_REM = OUT_ROWS - TAIL_BASE - TAIL_A
CH = 128
NEG = -3.0e38


def _topk_body(attn_ref, keep_ref, gidx_ref):
    a = attn_ref[0, :, 0, :]
    acc = a[0:1, :]
    for h in range(1, H):
        acc = acc + a[h:h + 1, :]
    sal = acc / jnp.float32(H)
    lane = lax.broadcasted_iota(jnp.int32, (1, S), 1)
    sal = jnp.where(lane == 0, NEG, sal)

    salT = jnp.transpose(jnp.broadcast_to(sal, (8, S)))[:, 0:1]

    i1 = lax.broadcasted_iota(jnp.int32, (CH, S), 1)
    rank_row = jnp.zeros((1, S), jnp.int32)
    for c in range(S // CH):
        salT_c = salT[c * CH:(c + 1) * CH, :]
        r0 = lax.broadcasted_iota(jnp.int32, (CH, S), 0) + c * CH
        gt = salT_c > sal
        eq = (salT_c == sal) & (r0 < i1)
        rank_row = rank_row + jnp.sum((gt | eq).astype(jnp.int32),
                                      axis=0, keepdims=True)

    rankT = jnp.transpose(jnp.broadcast_to(rank_row, (8, S)))[:, 0:1]

    p = lax.broadcasted_iota(jnp.int32, (CH, P_W), 1)
    keep = jnp.zeros((1, P_W), jnp.int32)
    gidx = jnp.zeros((1, P_W), jnp.int32)
    for c in range(S // CH):
        rankT_c = rankT[c * CH:(c + 1) * CH, :]
        r0 = lax.broadcasted_iota(jnp.int32, (CH, P_W), 0) + c * CH
        keep = keep + jnp.sum(jnp.where(rankT_c == p, r0, 0),
                              axis=0, keepdims=True)
        gidx = gidx + jnp.sum(jnp.where(rankT_c == p - 1, r0, 0),
                              axis=0, keepdims=True)
    keep_ref[...] = keep - 1
    gidx_ref[...] = gidx


def _topk_call(attn):
    return pl.pallas_call(
        _topk_body,
        grid=(1,),
        in_specs=[pl.BlockSpec((1, H, 8, S), lambda i: (0, 0, 0, 0))],
        out_specs=(pl.BlockSpec((1, P_W), lambda i: (0, 0)),
                   pl.BlockSpec((1, P_W), lambda i: (0, 0))),
        out_shape=(jax.ShapeDtypeStruct((1, P_W), jnp.int32),
                   jax.ShapeDtypeStruct((1, P_W), jnp.int32)),
    )(attn)


def _sc_gather_body(table_hbm, idx_hbm, out_hbm,
                    idx_v, rows_v, idx_a, rows_a, idx_b, rows_b, rows_r, sem):
    wid = lax.axis_index("s") * 2 + lax.axis_index("c")

    @pl.when(wid < N_FULL_W)
    def _full():
        base = wid * ROWS_PER_W
        pltpu.sync_copy(idx_hbm.at[pl.ds(base, ROWS_PER_W)], idx_v)
        pltpu.async_copy(table_hbm.at[idx_v], rows_v, sem).wait()
        pltpu.sync_copy(rows_v, out_hbm.at[pl.ds(base, ROWS_PER_W)])

    @pl.when(wid == N_FULL_W)
    def _tail():
        pltpu.sync_copy(idx_hbm.at[pl.ds(TAIL_BASE, TAIL_A)], idx_a)
        pltpu.async_copy(table_hbm.at[idx_a], rows_a, sem).wait()
        pltpu.sync_copy(rows_a, out_hbm.at[pl.ds(TAIL_BASE, TAIL_A)])
        pltpu.sync_copy(idx_hbm.at[pl.ds(TAIL_BASE + TAIL_A, TAIL_B)], idx_b)
        pltpu.async_copy(table_hbm.at[idx_b], rows_b, sem).wait()
        for r in range(TAIL_REM):
            for cs in range(0, D, 16):
                rows_r[r, cs:cs + 16] = rows_b[r, cs:cs + 16]
        pltpu.sync_copy(rows_r,
                        out_hbm.at[pl.ds(TAIL_BASE + TAIL_A, TAIL_REM)])


@functools.cache
def _sc_gather_call():
    mesh = plsc.VectorSubcoreMesh(core_axis_name="c", subcore_axis_name="s")
    return pl.kernel(
        _sc_gather_body,
        mesh=mesh,
        out_type=jax.ShapeDtypeStruct((OUT_ROWS, D), jnp.float32),
        scratch_types=[
            pltpu.VMEM((ROWS_PER_W,), jnp.int32),
            pltpu.VMEM((ROWS_PER_W, D), jnp.float32),
            pltpu.VMEM((TAIL_A,), jnp.int32),
            pltpu.VMEM((TAIL_A, D), jnp.float32),
            pltpu.VMEM((TAIL_B,), jnp.int32),
            pltpu.VMEM((TAIL_B, D), jnp.float32),
            pltpu.VMEM((TAIL_REM, D), jnp.float32),
            pltpu.SemaphoreType.DMA,
        ],
    )


def kernel(attn_score, x_unmasked):
    keep_full, gidx_full = _topk_call(attn_score)
    keep_timesteps = keep_full[:, :K]
    gidx = gidx_full.reshape(P_W)
    table = x_unmasked.reshape(S, D)
    rows = _sc_gather_call()(table, gidx)
    return (rows.reshape(1, OUT_ROWS, D), keep_timesteps)


# speedup vs baseline: 1.0710x; 1.0710x over previous
import functools

import jax
import jax.numpy as jnp
from jax import lax
from jax.experimental import pallas as pl
from jax.experimental.pallas import tpu as pltpu
from jax.experimental.pallas import tpu_sc as plsc

S = 2048
H = 12
D = 768
K = 1228
OUT_ROWS = K + 1
P_W = 1280
N_FULL_W = 30
ROWS_PER_W = 40
TAIL_BASE = N_FULL_W * ROWS_PER_W
TAIL_A = 24
TAIL_B = 8
TAIL_REM = OUT_ROWS - TAIL_BASE - TAIL_A
CH = 128
NEG = -3.0e38


def _topk_body(attn_ref, keep_ref, gidx_ref):
    a = attn_ref[0, :, 0, :]
    acc = a[0:1, :]
    for h in range(1, H):
        acc = acc + a[h:h + 1, :]
    sal = acc / jnp.float32(H)
    lane = lax.broadcasted_iota(jnp.int32, (1, S), 1)
    sal = jnp.where(lane == 0, NEG, sal)

    salT = jnp.transpose(jnp.broadcast_to(sal, (8, S)))[:, 0:1]

    i1 = lax.broadcasted_iota(jnp.int32, (CH, S), 1)
    rank_row = jnp.zeros((1, S), jnp.int32)
    for c in range(S // CH):
        salT_c = salT[c * CH:(c + 1) * CH, :]
        r0 = lax.broadcasted_iota(jnp.int32, (CH, S), 0) + c * CH
        gt = salT_c > sal
        eq = (salT_c == sal) & (r0 < i1)
        rank_row = rank_row + jnp.sum((gt | eq).astype(jnp.int32),
                                      axis=0, keepdims=True)

    rankT = jnp.transpose(jnp.broadcast_to(rank_row, (8, S)))[:, 0:1]

    p = lax.broadcasted_iota(jnp.int32, (CH, P_W), 1)
    keepj = jnp.zeros((1, P_W), jnp.int32)
    for c in range(S // CH):
        rankT_c = rankT[c * CH:(c + 1) * CH, :]
        r0 = lax.broadcasted_iota(jnp.int32, (CH, P_W), 0) + c * CH
        keepj = keepj + jnp.sum(jnp.where(rankT_c == p, r0, 0),
                                axis=0, keepdims=True)
    keep_ref[...] = (keepj - 1)[:, :K]
    lane = lax.broadcasted_iota(jnp.int32, (1, P_W), 1)
    gidx = jnp.where(lane == 0, 0, pltpu.roll(keepj, 1, 1))
    gidx_ref[...] = gidx.reshape(P_W)


def _topk_call(attn):
    return pl.pallas_call(
        _topk_body,
        grid=(1,),
        in_specs=[pl.BlockSpec((1, H, 8, S), lambda i: (0, 0, 0, 0))],
        out_specs=(pl.BlockSpec((1, K), lambda i: (0, 0)),
                   pl.BlockSpec((P_W,), lambda i: (0,))),
        out_shape=(jax.ShapeDtypeStruct((1, K), jnp.int32),
                   jax.ShapeDtypeStruct((P_W,), jnp.int32)),
    )(attn)


def _sc_gather_body(table_hbm, idx_hbm, out_hbm,
                    idx_v, rows_v, idx_a, rows_a, idx_b, rows_b, rows_r, sem):
    wid = lax.axis_index("s") * 2 + lax.axis_index("c")

    @pl.when(wid < N_FULL_W)
    def _full():
        base = wid * ROWS_PER_W
        pltpu.sync_copy(idx_hbm.at[pl.ds(base, ROWS_PER_W)], idx_v)
        pltpu.async_copy(table_hbm.at[idx_v], rows_v, sem).wait()
        pltpu.sync_copy(rows_v, out_hbm.at[pl.ds(base, ROWS_PER_W)])

    @pl.when(wid == N_FULL_W)
    def _tail():
        pltpu.sync_copy(idx_hbm.at[pl.ds(TAIL_BASE, TAIL_A)], idx_a)
        pltpu.async_copy(table_hbm.at[idx_a], rows_a, sem).wait()
        pltpu.sync_copy(rows_a, out_hbm.at[pl.ds(TAIL_BASE, TAIL_A)])
        pltpu.sync_copy(idx_hbm.at[pl.ds(TAIL_BASE + TAIL_A, TAIL_B)], idx_b)
        pltpu.async_copy(table_hbm.at[idx_b], rows_b, sem).wait()
        for r in range(TAIL_REM):
            for cs in range(0, D, 16):
                rows_r[r, cs:cs + 16] = rows_b[r, cs:cs + 16]
        pltpu.sync_copy(rows_r,
                        out_hbm.at[pl.ds(TAIL_BASE + TAIL_A, TAIL_REM)])


@functools.cache
def _sc_gather_call():
    mesh = plsc.VectorSubcoreMesh(core_axis_name="c", subcore_axis_name="s")
    return pl.kernel(
        _sc_gather_body,
        mesh=mesh,
        out_type=jax.ShapeDtypeStruct((OUT_ROWS, D), jnp.float32),
        scratch_types=[
            pltpu.VMEM((ROWS_PER_W,), jnp.int32),
            pltpu.VMEM((ROWS_PER_W, D), jnp.float32),
            pltpu.VMEM((TAIL_A,), jnp.int32),
            pltpu.VMEM((TAIL_A, D), jnp.float32),
            pltpu.VMEM((TAIL_B,), jnp.int32),
            pltpu.VMEM((TAIL_B, D), jnp.float32),
            pltpu.VMEM((TAIL_REM, D), jnp.float32),
            pltpu.SemaphoreType.DMA,
        ],
    )


def kernel(attn_score, x_unmasked):
    keep_timesteps, gidx = _topk_call(attn_score)
    table = x_unmasked.reshape(S, D)
    rows = _sc_gather_call()(table, gidx)
    return (rows.reshape(1, OUT_ROWS, D), keep_timesteps)
